# baseline (speedup 1.0000x reference)
import functools

import jax
import jax.numpy as jnp
from jax import lax
from jax.experimental import pallas as pl
from jax.experimental.pallas import tpu as pltpu
from jax.experimental.pallas import tpu_sc as plsc

VOCAB = 30522
D = 768
L = 128
B = 1024
BL = B * L
NW = 32
BPW = BL // NW
CH = 16
NCH = BPW // CH
NG = D // 16


def _make_kernel():
    mesh = plsc.VectorSubcoreMesh(core_axis_name="c", subcore_axis_name="s")

    @functools.partial(
        pl.kernel,
        mesh=mesh,
        out_type=jax.ShapeDtypeStruct((BL, D), jnp.float32),
        scratch_types=[
            pltpu.VMEM((BPW,), jnp.int32),
            pltpu.VMEM((L, D), jnp.float32),
            pltpu.VMEM((CH, D), jnp.float32),
            pltpu.SemaphoreType.DMA,
        ],
    )
    def emb_kernel(emb_hbm, idx_hbm, pos_hbm, out_hbm, idx_v, pos_v, buf, sem):
        wid = lax.axis_index("s") * 2 + lax.axis_index("c")
        base = wid * BPW
        pltpu.sync_copy(idx_hbm.at[pl.ds(base, BPW)], idx_v)
        pltpu.sync_copy(pos_hbm, pos_v)

        def chunk_body(c, carry):
            pltpu.async_copy(
                emb_hbm.at[idx_v.at[pl.ds(c * CH, CH)]], buf, sem
            ).wait()
            prow = (c % (L // CH)) * CH

            def row_body(j, carry2):
                for d in range(NG):
                    sl = pl.ds(d * 16, 16)
                    buf[j, sl] = buf[j, sl] + pos_v[prow + j, sl]
                return carry2

            lax.fori_loop(0, CH, row_body, 0, unroll=True)
            pltpu.sync_copy(buf, out_hbm.at[pl.ds(base + c * CH, CH)])
            return carry

        lax.fori_loop(0, NCH, chunk_body, 0)

    return emb_kernel


_emb_kernel = _make_kernel()


def kernel(input_ids, emb_weight, pos_weight):
    ids_flat = input_ids.reshape(BL).astype(jnp.int32)
    out = _emb_kernel(emb_weight, ids_flat, pos_weight)
    return out.reshape(B, L, D)


# speedup vs baseline: 5.3663x; 5.3663x over previous
import functools

import jax
import jax.numpy as jnp
from jax import lax
from jax.experimental import pallas as pl
from jax.experimental.pallas import tpu as pltpu
from jax.experimental.pallas import tpu_sc as plsc

VOCAB = 30522
D = 768
L = 128
B = 1024
BL = B * L
NW = 32
BPW = BL // NW
PPW = L // NW
CH = 16
NCH = BPW // CH
CPP = B // CH
NG = D // 16
NBUF = 4


def _make_kernel():
    mesh = plsc.VectorSubcoreMesh(core_axis_name="c", subcore_axis_name="s")

    @functools.partial(
        pl.kernel,
        mesh=mesh,
        out_type=jax.ShapeDtypeStruct((BL, D), jnp.float32),
        scratch_types=[
            pltpu.VMEM((BPW,), jnp.int32),
            pltpu.VMEM((PPW, D), jnp.float32),
            pltpu.VMEM((NBUF, CH, D), jnp.float32),
            pltpu.SemaphoreType.DMA,
            pltpu.SemaphoreType.DMA,
        ],
    )
    def emb_kernel(emb_hbm, idx_hbm, pos_hbm, out_hbm,
                   idx_v, pos_v, bufs, gsem, osem):
        wid = lax.axis_index("s") * 2 + lax.axis_index("c")
        base = wid * BPW
        pltpu.sync_copy(idx_hbm.at[pl.ds(base, BPW)], idx_v)
        pltpu.sync_copy(pos_hbm.at[pl.ds(wid * PPW, PPW)], pos_v)
        iota = lax.iota(jnp.int32, 16)

        def fire_gather(c, p):
            pltpu.async_copy(
                emb_hbm.at[idx_v.at[pl.ds(c * CH, CH)]], bufs.at[p], gsem)

        fire_gather(0, 0)
        fire_gather(1, 1)

        def outer_body(i, carry):
            c0 = i * NBUF
            for b in range(NBUF):
                c = c0 + b
                pltpu.make_async_copy(
                    emb_hbm.at[pl.ds(0, CH)], bufs.at[b], gsem).wait()
                nxt = (b + 2) % NBUF

                @pl.when(c >= 2)
                def _():
                    pltpu.make_async_copy(
                        emb_hbm.at[pl.ds(0, CH)], bufs.at[nxt], osem).wait()

                @pl.when(c + 2 < NCH)
                def _():
                    fire_gather(c + 2, nxt)

                lrow = c // CPP
                pos_regs = [pos_v[lrow, pl.ds(d * 16, 16)] for d in range(NG)]

                def row_body(j, carry2):
                    for d in range(NG):
                        sl = pl.ds(d * 16, 16)
                        bufs[b, j, sl] = bufs[b, j, sl] + pos_regs[d]
                    return carry2

                lax.fori_loop(0, CH, row_body, 0)

                pos_id = wid * PPW + lrow
                b0 = (c % CPP) * CH
                dst = (iota + b0) * L + pos_id
                pltpu.async_copy(bufs.at[b], out_hbm.at[dst], osem)
            return carry

        lax.fori_loop(0, NCH // NBUF, outer_body, 0)
        pltpu.make_async_copy(emb_hbm.at[pl.ds(0, CH)], bufs.at[2], osem).wait()
        pltpu.make_async_copy(emb_hbm.at[pl.ds(0, CH)], bufs.at[3], osem).wait()

    return emb_kernel


_emb_kernel = _make_kernel()


def kernel(input_ids, emb_weight, pos_weight):
    ids_t = input_ids.astype(jnp.int32).T.reshape(BL)
    out = _emb_kernel(emb_weight, ids_t, pos_weight)
    return out.reshape(B, L, D)
